# baseline (speedup 1.0000x reference)
import functools

import jax
import jax.numpy as jnp
from jax import lax
from jax.experimental import pallas as pl
from jax.experimental.pallas import tpu as pltpu, tpu_sc as plsc

N = 10000
E = 320000
F = 128
G = 128

NW = 32
NPAD = 10240
EPT = NPAD
EP = NW * EPT
C = 80
NCHUNK = EPT // C
NBUF = 2
RPT = NPAD // 16


def _agg_body(with_counts, h_hbm, src_hbm, dst_hbm, *refs):
  if with_counts:
    agg_hbm, cnt_hbm = refs[0], refs[1]
    refs = refs[2:]
  else:
    agg_hbm = refs[0]
    refs = refs[1:]
  (sidx_v, didx_v, rows_v, zbuf, ones_v, acc_sh, cnt_sh, isem, gsems,
   ssem) = refs

  c = lax.axis_index("c")
  s = lax.axis_index("s")
  w = c * 16 + s

  idx_cp = pltpu.async_copy(src_hbm.at[w], sidx_v, isem)
  didx_cp = pltpu.async_copy(dst_hbm.at[w], didx_v, isem)

  zv = jnp.zeros((16,), jnp.float32)
  for r in range(8):
    for k in range(F // 16):
      zbuf[r, pl.ds(k * 16, 16)] = zv

  zds = []
  for k in range(RPT // 8):
    zds.append(pltpu.async_copy(zbuf, acc_sh.at[pl.ds(s * RPT + k * 8, 8)],
                                ssem))
  if with_counts:
    for k in range(C // 16):
      ones_v[pl.ds(k * 16, 16)] = jnp.ones((16,), jnp.float32)
    for k in range(RPT // 128):
      zds.append(pltpu.async_copy(zbuf.at[0],
                                  cnt_sh.at[pl.ds(s * RPT + k * 128, 128)],
                                  ssem))
  for d in zds:
    d.wait()
  idx_cp.wait()
  didx_cp.wait()
  plsc.subcore_barrier()

  @pl.loop(0, NCHUNK // NBUF)
  def _(g):
    j0 = g * NBUF
    gds = []
    for i in range(NBUF):
      off = pl.multiple_of((j0 + i) * C, 8)
      gds.append(pltpu.async_copy(h_hbm.at[sidx_v.at[pl.ds(off, C)]],
                                  rows_v.at[i], gsems.at[i]))
    sds = []
    for i in range(NBUF):
      gds[i].wait()
      sds.append(pltpu.async_copy(rows_v.at[i], acc_sh.at[didx_v.at[j0 + i]],
                                  ssem, add=True))
      if with_counts:
        sds.append(pltpu.async_copy(ones_v, cnt_sh.at[didx_v.at[j0 + i]],
                                    ssem, add=True))
    for d in sds:
      d.wait()

  plsc.subcore_barrier()

  pltpu.sync_copy(acc_sh.at[pl.ds(s * RPT, RPT)],
                  agg_hbm.at[c].at[pl.ds(s * RPT, RPT)])
  if with_counts:
    pltpu.sync_copy(cnt_sh.at[pl.ds(s * RPT, RPT)],
                    cnt_hbm.at[c].at[pl.ds(s * RPT, RPT)])


@functools.lru_cache(maxsize=None)
def _make_agg(with_counts):
  out_type = [jax.ShapeDtypeStruct((2, NPAD, F), jnp.float32)]
  if with_counts:
    out_type.append(jax.ShapeDtypeStruct((2, NPAD), jnp.float32))
  return pl.kernel(
      functools.partial(_agg_body, with_counts),
      out_type=out_type,
      mesh=plsc.VectorSubcoreMesh(core_axis_name="c", subcore_axis_name="s",
                                  num_cores=2, num_subcores=16),
      scratch_types=[
          pltpu.VMEM((EPT,), jnp.int32),
          pltpu.VMEM((NCHUNK, C), jnp.int32),
          pltpu.VMEM((NBUF, C, F), jnp.float32),
          pltpu.VMEM((8, F), jnp.float32),
          pltpu.VMEM((C,), jnp.float32),
          pltpu.VMEM_SHARED((NPAD, F), jnp.float32),
          pltpu.VMEM_SHARED((NPAD,), jnp.float32),
          pltpu.SemaphoreType.DMA,
          pltpu.SemaphoreType.DMA((NBUF,)),
          pltpu.SemaphoreType.DMA,
      ],
  )


def _dense_body(agg_ref, cnt_ref, h_ref, wl_ref, bl_ref, wr_ref, g_ref,
                bb_ref, out_ref):
  agg = agg_ref[0, :N, :] + agg_ref[1, :N, :]
  cnt = cnt_ref[0, :N] + cnt_ref[1, :N]
  inv = 1.0 / jnp.maximum(cnt, 1.0)
  mean = agg * inv[:, None]
  t = (jnp.dot(mean, wl_ref[...], preferred_element_type=jnp.float32)
       + jnp.dot(h_ref[...], wr_ref[...], preferred_element_type=jnp.float32)
       + bl_ref[...])
  mu = jnp.mean(t, axis=0)
  xc = t - mu
  var = jnp.mean(xc * xc, axis=0)
  y = xc * (g_ref[...] * jax.lax.rsqrt(var + 1e-5)) + bb_ref[...]
  out_ref[...] = jnp.maximum(y, 0.0)


_dense = pl.pallas_call(
    _dense_body,
    out_shape=jax.ShapeDtypeStruct((N, F), jnp.float32),
)


def _pool_head_body(h_ref, b_ref, w1_ref, b1_ref, w2_ref, b2_ref, out_ref):
  h = h_ref[...]
  bvec = b_ref[...]
  gids = lax.broadcasted_iota(jnp.int32, (G, N), 0)
  oht = (gids == bvec).astype(jnp.float32)
  s = jnp.dot(oht, h, preferred_element_type=jnp.float32)
  cg = jnp.sum(oht, axis=1)
  pooled = s * (1.0 / jnp.maximum(cg, 1.0))[:, None]
  z = jnp.maximum(
      jnp.dot(pooled, w1_ref[...], preferred_element_type=jnp.float32)
      + b1_ref[...], 0.0)
  out_ref[...] = (jnp.dot(z, w2_ref[...], preferred_element_type=jnp.float32)
                  + b2_ref[...])


_pool_head = pl.pallas_call(
    _pool_head_body,
    out_shape=jax.ShapeDtypeStruct((G, 128), jnp.float32),
)


def kernel(x, ei, b, params):
  npad = EP - E
  src = jnp.concatenate([ei[0], jnp.zeros((npad,), jnp.int32)])
  pad_dst = N + jnp.arange(npad, dtype=jnp.int32) % (NPAD - N)
  dst = jnp.concatenate([ei[1], pad_dst])
  src2 = src.reshape(NW, EPT)
  dst3 = dst.reshape(NW, NCHUNK, C)
  h = x
  cnt = None
  for i in range(3):
    if cnt is None:
      agg2, cnt = _make_agg(True)(h, src2, dst3)
    else:
      (agg2,) = _make_agg(False)(h, src2, dst3)
    cp = params["convs"][i]
    bn = params["bns"][i]
    h = _dense(agg2, cnt, h, cp["Wl"].T, cp["bl"], cp["Wr"].T,
               bn["g"], bn["b"])
  hd = params["head"]
  return _pool_head(h, b.reshape(1, N), hd["W1"].T, hd["b1"], hd["W2"].T,
                    hd["b2"])


# speedup vs baseline: 2.9534x; 2.9534x over previous
import functools

import jax
import jax.numpy as jnp
from jax import lax
from jax.experimental import pallas as pl
from jax.experimental.pallas import tpu as pltpu, tpu_sc as plsc

N = 10000
E = 320000
F = 128
G = 128

NW = 32
NPAD = 10240
EPT = NPAD
EP = NW * EPT
C = 80
NCHUNK = EPT // C
NBUF = 2
RPT = NPAD // 16


def _agg_body(with_counts, h_hbm, src_hbm, dst_hbm, *refs):
  if with_counts:
    agg_hbm, cnt_hbm = refs[0], refs[1]
    refs = refs[2:]
  else:
    agg_hbm = refs[0]
    refs = refs[1:]
  (sidx_v, didx_v, rows_v, zbuf, ones_v, acc_sh, cnt_sh, isem, gsems,
   ssem) = refs

  c = lax.axis_index("c")
  s = lax.axis_index("s")
  w = c * 16 + s

  idx_cp = pltpu.async_copy(src_hbm.at[w], sidx_v, isem)
  didx_cp = pltpu.async_copy(dst_hbm.at[w], didx_v, isem)

  zv = jnp.zeros((16,), jnp.float32)
  for r in range(8):
    for k in range(F // 16):
      zbuf[r, pl.ds(k * 16, 16)] = zv

  zds = []
  for k in range(RPT // 8):
    zds.append(pltpu.async_copy(zbuf, acc_sh.at[pl.ds(s * RPT + k * 8, 8)],
                                ssem))
  if with_counts:
    for k in range(C // 16):
      ones_v[pl.ds(k * 16, 16)] = jnp.ones((16,), jnp.float32)
    for k in range(RPT // 128):
      zds.append(pltpu.async_copy(zbuf.at[0],
                                  cnt_sh.at[pl.ds(s * RPT + k * 128, 128)],
                                  ssem))
  for d in zds:
    d.wait()
  idx_cp.wait()
  didx_cp.wait()
  plsc.subcore_barrier()

  @pl.loop(0, NCHUNK // NBUF)
  def _(g):
    j0 = g * NBUF
    gds = []
    for i in range(NBUF):
      off = pl.multiple_of((j0 + i) * C, 8)
      gds.append(pltpu.async_copy(h_hbm.at[sidx_v.at[pl.ds(off, C)]],
                                  rows_v.at[i], gsems.at[i]))
    sds = []
    for i in range(NBUF):
      gds[i].wait()
      sds.append(pltpu.async_copy(rows_v.at[i], acc_sh.at[didx_v.at[j0 + i]],
                                  ssem, add=True))
      if with_counts:
        sds.append(pltpu.async_copy(ones_v, cnt_sh.at[didx_v.at[j0 + i]],
                                    ssem, add=True))
    for d in sds:
      d.wait()

  plsc.subcore_barrier()

  pltpu.sync_copy(acc_sh.at[pl.ds(s * RPT, RPT)],
                  agg_hbm.at[c].at[pl.ds(s * RPT, RPT)])
  if with_counts:
    pltpu.sync_copy(cnt_sh.at[pl.ds(s * RPT, RPT)],
                    cnt_hbm.at[c].at[pl.ds(s * RPT, RPT)])


@functools.lru_cache(maxsize=None)
def _make_agg(with_counts):
  out_type = [jax.ShapeDtypeStruct((2, NPAD, F), jnp.float32)]
  if with_counts:
    out_type.append(jax.ShapeDtypeStruct((2, NPAD), jnp.float32))
  return pl.kernel(
      functools.partial(_agg_body, with_counts),
      out_type=out_type,
      mesh=plsc.VectorSubcoreMesh(core_axis_name="c", subcore_axis_name="s",
                                  num_cores=2, num_subcores=16),
      scratch_types=[
          pltpu.VMEM((EPT,), jnp.int32),
          pltpu.VMEM((NCHUNK, C), jnp.int32),
          pltpu.VMEM((NBUF, C, F), jnp.float32),
          pltpu.VMEM((8, F), jnp.float32),
          pltpu.VMEM((C,), jnp.float32),
          pltpu.VMEM_SHARED((NPAD, F), jnp.float32),
          pltpu.VMEM_SHARED((NPAD,), jnp.float32),
          pltpu.SemaphoreType.DMA,
          pltpu.SemaphoreType.DMA((NBUF,)),
          pltpu.SemaphoreType.DMA,
      ],
  )


def _dense_body(agg_ref, cnt_ref, h_ref, wl_ref, bl_ref, wr_ref, g_ref,
                bb_ref, out_ref):
  agg = agg_ref[0, :N, :] + agg_ref[1, :N, :]
  cnt = cnt_ref[0, :N] + cnt_ref[1, :N]
  inv = 1.0 / jnp.maximum(cnt, 1.0)
  mean = agg * inv[:, None]
  t = (jnp.dot(mean, wl_ref[...], preferred_element_type=jnp.float32)
       + jnp.dot(h_ref[...], wr_ref[...], preferred_element_type=jnp.float32)
       + bl_ref[...])
  mu = jnp.mean(t, axis=0)
  xc = t - mu
  var = jnp.mean(xc * xc, axis=0)
  y = xc * (g_ref[...] * jax.lax.rsqrt(var + 1e-5)) + bb_ref[...]
  out_ref[...] = jnp.maximum(y, 0.0)


_dense = pl.pallas_call(
    _dense_body,
    out_shape=jax.ShapeDtypeStruct((N, F), jnp.float32),
)


def _pool_head_body(h_ref, b_ref, w1_ref, b1_ref, w2_ref, b2_ref, out_ref):
  h = h_ref[...]
  bvec = b_ref[...]
  gids = lax.broadcasted_iota(jnp.int32, (G, N), 0)
  oht = (gids == bvec).astype(jnp.float32)
  s = jnp.dot(oht, h, preferred_element_type=jnp.float32)
  cg = jnp.sum(oht, axis=1)
  pooled = s * (1.0 / jnp.maximum(cg, 1.0))[:, None]
  z = jnp.maximum(
      jnp.dot(pooled, w1_ref[...], preferred_element_type=jnp.float32)
      + b1_ref[...], 0.0)
  out_ref[...] = (jnp.dot(z, w2_ref[...], preferred_element_type=jnp.float32)
                  + b2_ref[...])


_pool_head = pl.pallas_call(
    _pool_head_body,
    out_shape=jax.ShapeDtypeStruct((G, 128), jnp.float32),
)


def kernel(x, ei, b, params):
  npad = EP - E
  pad_src = jnp.arange(npad, dtype=jnp.int32) * 13 % N
  src = jnp.concatenate([ei[0], pad_src])
  pad_dst = N + jnp.arange(npad, dtype=jnp.int32) % (NPAD - N)
  dst = jnp.concatenate([ei[1], pad_dst])
  src2 = src.reshape(NW, EPT)
  dst3 = dst.reshape(NW, NCHUNK, C)
  h = x
  cnt = None
  for i in range(3):
    if cnt is None:
      agg2, cnt = _make_agg(True)(h, src2, dst3)
    else:
      (agg2,) = _make_agg(False)(h, src2, dst3)
    cp = params["convs"][i]
    bn = params["bns"][i]
    h = _dense(agg2, cnt, h, cp["Wl"].T, cp["bl"], cp["Wr"].T,
               bn["g"], bn["b"])
  hd = params["head"]
  return _pool_head(h, b.reshape(1, N), hd["W1"].T, hd["b1"], hd["W2"].T,
                    hd["b2"])
